# baseline (speedup 1.0000x reference)
import functools

import jax
import jax.numpy as jnp
from jax import lax
from jax.experimental import pallas as pl
from jax.experimental.pallas import tpu as pltpu
from jax.experimental.pallas import tpu_sc as plsc

_NC = 2
_NS = 16
_NW = _NC * _NS
_LANES = 16

_CHUNK = 1024


def _sc_gather(args, B, V, D, n_groups):
  per_w = B // _NW
  assert n_groups % 2 == 0 and n_groups >= 4
  mesh = plsc.VectorSubcoreMesh(core_axis_name="c", subcore_axis_name="s")

  @functools.partial(
      pl.kernel,
      out_type=jax.ShapeDtypeStruct((B * D,), jnp.float32),
      mesh=mesh,
      scratch_types=[
          pltpu.VMEM((V * D,), jnp.float32),
          pltpu.VMEM((2, _CHUNK), jnp.int32),
          pltpu.VMEM((2, _CHUNK * D), jnp.float32),
          pltpu.VMEM((_LANES,), jnp.int32),
          pltpu.SemaphoreType.DMA,
          pltpu.SemaphoreType.DMA,
          pltpu.SemaphoreType.DMA,
          pltpu.SemaphoreType.DMA,
      ],
      compiler_params=pltpu.CompilerParams(use_tc_tiling_on_sc=False,
                                           needs_layout_passes=False),
  )
  def k(idx_hbm, off_hbm, table_hbm, out_hbm, table_v, idx_v, rows_v, off_v,
        sem_i0, sem_i1, sem_o0, sem_o1):
    wid = lax.axis_index("s") * _NC + lax.axis_index("c")
    base = wid * per_w
    sem_i = (sem_i0, sem_i1)
    sem_o = (sem_o0, sem_o1)
    pltpu.sync_copy(off_hbm, off_v)
    pltpu.sync_copy(table_hbm, table_v)
    offv = off_v[...] * D
    iota = lax.iota(jnp.int32, _LANES)
    iota_d = iota * D
    one = jnp.full((_LANES,), 1, dtype=jnp.int32)

    pltpu.async_copy(idx_hbm.at[pl.ds(base, _CHUNK)], idx_v.at[0], sem_i[0])
    pltpu.async_copy(idx_hbm.at[pl.ds(base + _CHUNK, _CHUNK)], idx_v.at[1],
                     sem_i[1])

    def gather_group(p):
      iv = idx_v.at[p]
      rv = rows_v.at[p]

      def body(c4, carry):
        idxv = plsc.load_gather(iv, [c4 * _LANES + iota])
        eidx = idxv * D + offv
        sidx = iota_d + jnp.full((_LANES,), c4 * (_LANES * D), dtype=jnp.int32)
        for _ in range(D):
          g = plsc.load_gather(table_v, [eidx])
          plsc.store_scatter(rv, [sidx], g)
          eidx = eidx + one
          sidx = sidx + one
        return carry

      lax.fori_loop(0, _CHUNK // _LANES, body, None)

    def half(g2, p):
      g = g2 * 2 + p
      start = base + g * _CHUNK
      rv = rows_v.at[p]
      out_slice = out_hbm.at[pl.ds(start * D, _CHUNK * D)]

      @pl.when(g2 >= 1)
      def _():
        pltpu.make_async_copy(rv, out_slice, sem_o[p]).wait()

      pltpu.make_async_copy(idx_hbm.at[pl.ds(start, _CHUNK)], idx_v.at[p],
                            sem_i[p]).wait()
      gather_group(p)

      pltpu.async_copy(rv, out_slice, sem_o[p])

      @pl.when(g2 < n_groups // 2 - 1)
      def _():
        pltpu.async_copy(idx_hbm.at[pl.ds(start + 2 * _CHUNK, _CHUNK)],
                         idx_v.at[p], sem_i[p])

    def pair(g2, carry):
      half(g2, 0)
      half(g2, 1)
      return carry

    lax.fori_loop(0, n_groups // 2, pair, None)

    tail = base + (n_groups - 2) * _CHUNK
    pltpu.make_async_copy(rows_v.at[0],
                          out_hbm.at[pl.ds(tail * D, _CHUNK * D)],
                          sem_o[0]).wait()
    pltpu.make_async_copy(rows_v.at[1],
                          out_hbm.at[pl.ds((tail + _CHUNK) * D, _CHUNK * D)],
                          sem_o[1]).wait()

  idx, off_vec, table = args
  return k(idx, off_vec, table)


def kernel(rel_pos_matrix, len, embeddings_table):
  n = rel_pos_matrix.shape[0]
  V, D = embeddings_table.shape
  B = n * n
  idx = rel_pos_matrix.reshape(B)
  off = jnp.asarray(len, jnp.int32) - jnp.int32(n)
  off_vec = jnp.full((_LANES,), off, dtype=jnp.int32)
  per_w = B // _NW
  assert per_w % _CHUNK == 0
  out = _sc_gather((idx, off_vec, embeddings_table.reshape(V * D)), B, V, D,
                   per_w // _CHUNK)
  return out.reshape(n, n, D)


# speedup vs baseline: 1.2639x; 1.2639x over previous
import functools

import jax
import jax.numpy as jnp
from jax import lax
from jax.experimental import pallas as pl
from jax.experimental.pallas import tpu as pltpu
from jax.experimental.pallas import tpu_sc as plsc

_NC = 2
_NS = 16
_NW = _NC * _NS
_LANES = 16

_CHUNK = 1024


def _sc_gather(args, B, V, D, n_groups):
  per_w = B // _NW
  assert n_groups % 2 == 0 and n_groups >= 4
  mesh = plsc.VectorSubcoreMesh(core_axis_name="c", subcore_axis_name="s")

  @functools.partial(
      pl.kernel,
      out_type=jax.ShapeDtypeStruct((B * D,), jnp.float32),
      mesh=mesh,
      scratch_types=[
          pltpu.VMEM((V * D,), jnp.float32),
          pltpu.VMEM((2, _CHUNK), jnp.int32),
          pltpu.VMEM((2, _CHUNK * D), jnp.float32),
          pltpu.VMEM((_LANES,), jnp.int32),
          pltpu.SemaphoreType.DMA,
          pltpu.SemaphoreType.DMA,
          pltpu.SemaphoreType.DMA,
          pltpu.SemaphoreType.DMA,
      ],
      compiler_params=pltpu.CompilerParams(use_tc_tiling_on_sc=False,
                                           needs_layout_passes=False),
  )
  def k(idx_hbm, off_hbm, table_hbm, out_hbm, table_v, idx_v, rows_v, off_v,
        sem_i0, sem_i1, sem_o0, sem_o1):
    wid = lax.axis_index("s") * _NC + lax.axis_index("c")
    base = wid * per_w
    sem_i = (sem_i0, sem_i1)
    sem_o = (sem_o0, sem_o1)
    pltpu.sync_copy(off_hbm, off_v)
    pltpu.sync_copy(table_hbm, table_v)
    offv = off_v[...] * D
    iota = lax.iota(jnp.int32, _LANES)
    iota_d = iota * D
    one = jnp.full((_LANES,), 1, dtype=jnp.int32)

    pltpu.async_copy(idx_hbm.at[pl.ds(base, _CHUNK)], idx_v.at[0], sem_i[0])
    pltpu.async_copy(idx_hbm.at[pl.ds(base + _CHUNK, _CHUNK)], idx_v.at[1],
                     sem_i[1])

    def gather_group(p):
      iv = idx_v.at[p]
      rv = rows_v.at[p]

      def body(c4, carry):
        idxv = plsc.load_gather(iv, [c4 * _LANES + iota])
        base_e = idxv * D + offv
        base_s = iota_d + jnp.full((_LANES,), c4 * (_LANES * D),
                                   dtype=jnp.int32)
        gs = []
        eidx = base_e
        for _ in range(D):
          gs.append(plsc.load_gather(table_v, [eidx]))
          eidx = eidx + one
        sidx = base_s
        for d in range(D):
          plsc.store_scatter(rv, [sidx], gs[d])
          sidx = sidx + one
        return carry

      lax.fori_loop(0, _CHUNK // _LANES, body, None)

    def half(g2, p):
      g = g2 * 2 + p
      start = base + g * _CHUNK
      rv = rows_v.at[p]
      out_slice = out_hbm.at[pl.ds(start * D, _CHUNK * D)]

      @pl.when(g2 >= 1)
      def _():
        pltpu.make_async_copy(rv, out_slice, sem_o[p]).wait()

      pltpu.make_async_copy(idx_hbm.at[pl.ds(start, _CHUNK)], idx_v.at[p],
                            sem_i[p]).wait()
      gather_group(p)

      pltpu.async_copy(rv, out_slice, sem_o[p])

      @pl.when(g2 < n_groups // 2 - 1)
      def _():
        pltpu.async_copy(idx_hbm.at[pl.ds(start + 2 * _CHUNK, _CHUNK)],
                         idx_v.at[p], sem_i[p])

    def pair(g2, carry):
      half(g2, 0)
      half(g2, 1)
      return carry

    lax.fori_loop(0, n_groups // 2, pair, None)

    tail = base + (n_groups - 2) * _CHUNK
    pltpu.make_async_copy(rows_v.at[0],
                          out_hbm.at[pl.ds(tail * D, _CHUNK * D)],
                          sem_o[0]).wait()
    pltpu.make_async_copy(rows_v.at[1],
                          out_hbm.at[pl.ds((tail + _CHUNK) * D, _CHUNK * D)],
                          sem_o[1]).wait()

  idx, off_vec, table = args
  return k(idx, off_vec, table)


def kernel(rel_pos_matrix, len, embeddings_table):
  n = rel_pos_matrix.shape[0]
  V, D = embeddings_table.shape
  B = n * n
  idx = rel_pos_matrix.reshape(B)
  off = jnp.asarray(len, jnp.int32) - jnp.int32(n)
  off_vec = jnp.full((_LANES,), off, dtype=jnp.int32)
  per_w = B // _NW
  assert per_w % _CHUNK == 0
  out = _sc_gather((idx, off_vec, embeddings_table.reshape(V * D)), B, V, D,
                   per_w // _CHUNK)
  return out.reshape(n, n, D)


# speedup vs baseline: 2.9877x; 2.3639x over previous
import functools

import jax
import jax.numpy as jnp
from jax import lax
from jax.experimental import pallas as pl
from jax.experimental.pallas import tpu as pltpu
from jax.experimental.pallas import tpu_sc as plsc

_NC = 2
_NS = 16
_NW = _NC * _NS
_LANES = 16

_CHUNK = 1024


def _vtake(v, ids):
  return lax.gather(
      v, ids[:, None],
      dimension_numbers=lax.GatherDimensionNumbers(
          offset_dims=(), collapsed_slice_dims=(0,), start_index_map=(0,)),
      slice_sizes=(1,),
      mode=lax.GatherScatterMode.PROMISE_IN_BOUNDS)


def _sc_gather(args, B, V, D, n_groups):
  per_w = B // _NW
  assert n_groups % 2 == 0 and n_groups >= 4
  mesh = plsc.VectorSubcoreMesh(core_axis_name="c", subcore_axis_name="s")

  @functools.partial(
      pl.kernel,
      out_type=jax.ShapeDtypeStruct((B * D,), jnp.float32),
      mesh=mesh,
      scratch_types=[
          pltpu.VMEM((V * D,), jnp.float32),
          pltpu.VMEM((2, _CHUNK), jnp.int32),
          pltpu.VMEM((2, _CHUNK * D), jnp.float32),
          pltpu.VMEM((_LANES,), jnp.int32),
          pltpu.SemaphoreType.DMA,
          pltpu.SemaphoreType.DMA,
          pltpu.SemaphoreType.DMA,
          pltpu.SemaphoreType.DMA,
      ],
      compiler_params=pltpu.CompilerParams(use_tc_tiling_on_sc=False,
                                           needs_layout_passes=False),
  )
  def k(idx_hbm, off_hbm, table_hbm, out_hbm, table_v, idx_v, rows_v, off_v,
        sem_i0, sem_i1, sem_o0, sem_o1):
    wid = lax.axis_index("s") * _NC + lax.axis_index("c")
    base = wid * per_w
    sem_i = (sem_i0, sem_i1)
    sem_o = (sem_o0, sem_o1)
    pltpu.sync_copy(off_hbm, off_v)
    pltpu.sync_copy(table_hbm, table_v)
    offv = off_v[...] * D
    iota = lax.iota(jnp.int32, _LANES)
    oi = offv + iota
    consts = [jnp.full((_LANES,), h * _LANES, dtype=jnp.int32)
              for h in range(D // _LANES)]

    pltpu.async_copy(idx_hbm.at[pl.ds(base, _CHUNK)], idx_v.at[0], sem_i[0])
    pltpu.async_copy(idx_hbm.at[pl.ds(base + _CHUNK, _CHUNK)], idx_v.at[1],
                     sem_i[1])

    def gather_group(p):
      iv = idx_v.at[p]
      rv = rows_v.at[p]

      def body(c4, carry):
        idxv = plsc.load_gather(iv, [c4 * _LANES + iota])
        gs = []
        for r in range(_LANES):
          ridx = _vtake(idxv, jnp.full((_LANES,), r, dtype=jnp.int32))
          e0 = ridx * D + oi
          for h in range(D // _LANES):
            gs.append(plsc.load_gather(table_v, [e0 + consts[h]]))
        base_w = c4 * (_LANES * D)
        for j, g in enumerate(gs):
          rv[pl.ds(base_w + j * _LANES, _LANES)] = g
        return carry

      lax.fori_loop(0, _CHUNK // _LANES, body, None)

    def half(g2, p):
      g = g2 * 2 + p
      start = base + g * _CHUNK
      rv = rows_v.at[p]
      out_slice = out_hbm.at[pl.ds(start * D, _CHUNK * D)]

      @pl.when(g2 >= 1)
      def _():
        pltpu.make_async_copy(rv, out_slice, sem_o[p]).wait()

      pltpu.make_async_copy(idx_hbm.at[pl.ds(start, _CHUNK)], idx_v.at[p],
                            sem_i[p]).wait()
      gather_group(p)

      pltpu.async_copy(rv, out_slice, sem_o[p])

      @pl.when(g2 < n_groups // 2 - 1)
      def _():
        pltpu.async_copy(idx_hbm.at[pl.ds(start + 2 * _CHUNK, _CHUNK)],
                         idx_v.at[p], sem_i[p])

    def pair(g2, carry):
      half(g2, 0)
      half(g2, 1)
      return carry

    lax.fori_loop(0, n_groups // 2, pair, None)

    tail = base + (n_groups - 2) * _CHUNK
    pltpu.make_async_copy(rows_v.at[0],
                          out_hbm.at[pl.ds(tail * D, _CHUNK * D)],
                          sem_o[0]).wait()
    pltpu.make_async_copy(rows_v.at[1],
                          out_hbm.at[pl.ds((tail + _CHUNK) * D, _CHUNK * D)],
                          sem_o[1]).wait()

  idx, off_vec, table = args
  return k(idx, off_vec, table)


def kernel(rel_pos_matrix, len, embeddings_table):
  n = rel_pos_matrix.shape[0]
  V, D = embeddings_table.shape
  B = n * n
  idx = rel_pos_matrix.reshape(B)
  off = jnp.asarray(len, jnp.int32) - jnp.int32(n)
  off_vec = jnp.full((_LANES,), off, dtype=jnp.int32)
  per_w = B // _NW
  assert per_w % _CHUNK == 0
  out = _sc_gather((idx, off_vec, embeddings_table.reshape(V * D)), B, V, D,
                   per_w // _CHUNK)
  return out.reshape(n, n, D)
